# baseline (speedup 1.0000x reference)
import functools

import jax
import jax.numpy as jnp
from jax.experimental import pallas as pl
from jax.experimental.pallas import tpu as pltpu
from jax.experimental.pallas import tpu_sc as plsc

WS = 7
NH = 8
NP = 4
NL = 4
D = 256
HD = D // NH
DFF = 1024
BS = 2
SHAPES = ((64, 64), (32, 32), (16, 16), (8, 8))
LQ = sum(h * w for h, w in SHAPES)
START = (0, 4096, 5120, 5376, 5440)
HALO = 4
PADS = tuple((h + 2 * HALO, w + 2 * HALO) for h, w in SHAPES)
PHW = tuple(ph * pw for ph, pw in PADS)
LVL_BASE = (0,)
for _p in PHW:
    LVL_BASE = LVL_BASE + (LVL_BASE[-1] + BS * NH * _p,)
TOT_ROWS = LVL_BASE[-1]
NT = NH * NL * NP * 4
NR = NH * NL * NP
QB = 256
RB = 256


def _topk_body(mp_ref, sv_ref, si_ref, *, H, W, HB):
    base = pl.program_id(1) * HB
    sub = mp_ref[0, pl.ds(base, HB + 6), :]
    mx = [sub[:, dx:dx + W] for dx in range(WS)]
    ninf = jnp.full((HB, W), -jnp.inf, jnp.float32)
    zi = jnp.zeros((HB, W), jnp.int32)
    bv = [ninf] * (2 * NP)
    bi = [zi] * (2 * NP)
    for s in range(WS * WS):
        dy, dx = s // WS, s % WS
        if True:
            shift = mx[dx][dy:dy + HB, :]
            for sgn in range(2):
                v = shift if sgn == 0 else -shift
                i = jnp.full((HB, W), s, jnp.int32)
                for k in range(NP):
                    o = sgn * NP + k
                    gt = v > bv[o]
                    bv[o], v = jnp.where(gt, v, bv[o]), jnp.where(gt, bv[o], v)
                    bi[o], i = jnp.where(gt, i, bi[o]), jnp.where(gt, bi[o], i)
    sv_ref[...] = jnp.stack(bv, axis=0)[None]
    si_ref[...] = jnp.stack(bi, axis=0)[None]


def _run_topk(mp, H, W):
    HB = min(H, 16)
    grid = (BS, H // HB)
    return pl.pallas_call(
        functools.partial(_topk_body, H=H, W=W, HB=HB),
        grid=grid,
        in_specs=[pl.BlockSpec((1, H + 6, W + 6), lambda b, j: (b, 0, 0))],
        out_specs=(pl.BlockSpec((1, 2 * NP, HB, W), lambda b, j: (b, 0, j, 0)),
                   pl.BlockSpec((1, 2 * NP, HB, W), lambda b, j: (b, 0, j, 0))),
        out_shape=(jax.ShapeDtypeStruct((BS, 2 * NP, H, W), jnp.float32),
                   jax.ShapeDtypeStruct((BS, 2 * NP, H, W), jnp.int32)),
    )(mp)


def _coef_body(sv_ref, si_ref, ref_ref, pw_ref, gidx_ref, coef_ref):
    b = pl.program_id(0)
    j = pl.program_id(1)
    lam = ((j >= 16).astype(jnp.int32) + (j >= 20).astype(jnp.int32)
           + (j >= 21).astype(jnp.int32))
    wsel = jnp.zeros((NH, 1), jnp.float32)
    for l in range(NL):
        wsel = jnp.where(lam == l, pw_ref[:, l:l + 1], wsel)
    pos = wsel >= 0.0
    sv = sv_ref[0]
    si = si_ref[0]
    vals = jnp.where(pos[:, :, None], sv[None, 0:NP], sv[None, NP:2 * NP])
    idx = jnp.where(pos[:, :, None], si[None, 0:NP], si[None, NP:2 * NP])
    z = jnp.abs(wsel)[:, :, None] * vals
    zm = jnp.maximum(jnp.maximum(z[:, 0], z[:, 1]),
                     jnp.maximum(z[:, 2], z[:, 3]))
    e = jnp.exp(z - zm[:, None, :])
    es = e[:, 0] + e[:, 1] + e[:, 2] + e[:, 3]
    attn = e / (NL * es)[:, None, :]
    ref2 = ref_ref[0]
    hiota = jax.lax.broadcasted_iota(jnp.int32, (NH, 1, 1), 0)
    rows_l = []
    coef_l = []
    for l, (H, W) in enumerate(SHAPES):
        PH, PW = PADS[l]
        x = ref2[2 * l] * W - 0.5
        y = ref2[2 * l + 1] * H - 0.5
        x0f = jnp.floor(x)
        y0f = jnp.floor(y)
        lx = x - x0f
        ly = y - y0f
        off = (idx // WS) * PW + (idx % WS)
        qbase = (y0f.astype(jnp.int32) * PW + x0f.astype(jnp.int32)
                 + (LVL_BASE[l] + b * NH * PHW[l]
                    + (HALO - WS // 2) * PW + (HALO - WS // 2)))
        r00 = off + qbase[None, None, :] + hiota * PHW[l]
        rows_l.append(jnp.clip(r00, 0, TOT_ROWS - 1))
        ca = attn
        coef_l.append(jnp.stack(
            [ca * ((1 - lx) * (1 - ly))[None, None, :],
             ca * (lx * (1 - ly))[None, None, :],
             ca * ((1 - lx) * ly)[None, None, :],
             ca * (lx * ly)[None, None, :]], axis=2))
    rows = jnp.stack(rows_l, axis=1)
    coefs = jnp.stack(coef_l, axis=1)
    gidx_ref[...] = rows.reshape(1, NR, QB)
    coef_ref[...] = coefs.reshape(1, NT, QB)


def _run_coef(sv, si, refT, parse_wT):
    nblk = (LQ + QB - 1) // QB
    grid = (BS, nblk)
    return pl.pallas_call(
        _coef_body,
        grid=grid,
        in_specs=[
            pl.BlockSpec((1, 2 * NP, QB), lambda b, j: (b, 0, j)),
            pl.BlockSpec((1, 2 * NP, QB), lambda b, j: (b, 0, j)),
            pl.BlockSpec((1, 2 * NL, QB), lambda b, j: (b, 0, j)),
            pl.BlockSpec((NH, NL), lambda b, j: (0, 0)),
        ],
        out_specs=(
            pl.BlockSpec((1, NR, QB), lambda b, j: (b, 0, j)),
            pl.BlockSpec((1, NT, QB), lambda b, j: (b, 0, j)),
        ),
        out_shape=(jax.ShapeDtypeStruct((BS, NR, LQ), jnp.int32),
                   jax.ShapeDtypeStruct((BS, NT, LQ), jnp.float32)),
    )(sv, si, refT, parse_wT)


def _val_body(x_ref, w_ref, b_ref, o_ref):
    o_ref[...] = (jnp.dot(x_ref[...], w_ref[...],
                          preferred_element_type=jnp.float32) + b_ref[...])


def _run_val(src2d, Wv, bv):
    n = src2d.shape[0]
    grid = ((n + RB - 1) // RB,)
    return pl.pallas_call(
        _val_body,
        grid=grid,
        in_specs=[
            pl.BlockSpec((RB, D), lambda i: (i, 0)),
            pl.BlockSpec((D, D), lambda i: (0, 0)),
            pl.BlockSpec((1, D), lambda i: (0, 0)),
        ],
        out_specs=pl.BlockSpec((RB, D), lambda i: (i, 0)),
        out_shape=jax.ShapeDtypeStruct((n, D), jnp.float32),
    )(src2d, Wv, bv)


NWORK = 32
QPW = BS * LQ // NWORK
CQ = 2
NCHUNK = QPW // CQ


CN = CQ * NT
CR = CQ * NR


def _sc_body(table, idxr, coefr, out_hbm, idxv, coefv, rowsv, outv, sem):
    cid = jax.lax.axis_index("c")
    sid = jax.lax.axis_index("s")
    wid = sid * 2 + cid
    q0w = wid * QPW

    def load_fire(t, pb):
        q0 = q0w + t * CQ
        pltpu.sync_copy(idxr.at[pl.ds(q0, CQ)],
                        idxv.at[pl.ds(pb * CQ, CQ)])
        pltpu.sync_copy(coefr.at[pl.ds(q0 * NT, CN)],
                        coefv.at[pl.ds(pb * CN, CN)])
        for jj in range(CQ):
            pltpu.async_copy(
                table.at[idxv.at[pb * CQ + jj]],
                rowsv.at[pl.ds(pb * CR + jj * NR, NR)], sem)

    def drain(pb):
        for jj in range(CQ):
            pltpu.make_async_copy(
                table.at[idxv.at[pb * CQ + jj]],
                rowsv.at[pl.ds(pb * CR + jj * NR, NR)], sem).wait()

    load_fire(0, 0)

    def chunk(t, carry):
        pb = jax.lax.rem(t, 2)
        drain(pb)
        tn = jnp.minimum(t + 1, NCHUNK - 1)
        load_fire(tn, 1 - pb)
        cb = pb * CN
        rb0 = pb * CR
        for qq in range(CQ):
            for h in range(NH):
                def body(bk, ac, qq=qq, h=h, cb=cb, rb0=rb0):
                    ac = list(ac)
                    cbase = cb + qq * NT + h * 64 + bk * 16
                    rbase = rb0 + qq * NR + h * 16 + bk * 4
                    for j2 in range(4):
                        for cc in range(4):
                            c = plsc.load_gather(
                                coefv, [jnp.zeros((16,), jnp.int32)
                                        + (cbase + j2 * 4 + cc)])
                            w = plsc.bitcast(
                                rowsv[rbase + j2, pl.ds(cc * HD, HD)],
                                jnp.int32)
                            r0 = plsc.bitcast(w << 16, jnp.float32)
                            r1 = plsc.bitcast(w & jnp.int32(-65536),
                                              jnp.float32)
                            ac[2 * cc] = ac[2 * cc] + c * r0
                            ac[2 * cc + 1] = ac[2 * cc + 1] + c * r1
                    return tuple(ac)

                z16 = jnp.zeros((16,), jnp.float32)
                acc = jax.lax.fori_loop(0, 4, body, (z16,) * 8)
                outv[pb * CQ + qq, pl.ds(h * HD, 16)] = (
                    (acc[0] + acc[2]) + (acc[4] + acc[6]))
                outv[pb * CQ + qq, pl.ds(h * HD + 16, 16)] = (
                    (acc[1] + acc[3]) + (acc[5] + acc[7]))
        pltpu.sync_copy(outv.at[pl.ds(pb * CQ, CQ)],
                        out_hbm.at[pl.ds(q0w + t * CQ, CQ)])
        return carry

    jax.lax.fori_loop(0, NCHUNK, chunk, 0)
    drain(0)


@functools.lru_cache(maxsize=1)
def _sc_gather_fn():
    return functools.partial(
        pl.kernel,
        out_type=jax.ShapeDtypeStruct((BS * LQ, D), jnp.float32),
        mesh=plsc.VectorSubcoreMesh(core_axis_name="c", subcore_axis_name="s"),
        compiler_params=pltpu.CompilerParams(use_tc_tiling_on_sc=False,
                                             needs_layout_passes=False),
        scratch_types=[
            pltpu.VMEM((2 * CQ, NR), jnp.int32),
            pltpu.VMEM((2 * CN,), jnp.float32),
            pltpu.VMEM((2 * CR, 4 * HD), jnp.bfloat16),
            pltpu.VMEM((2 * CQ, D), jnp.float32),
            pltpu.SemaphoreType.DMA,
        ],
    )(_sc_body)


def _sc_gather(table, gidx_q, coef_q):
    return _sc_gather_fn()(table, gidx_q, coef_q)


def _tail_body(at_ref, src_ref, wo_ref, bo_ref, g1_ref, be1_ref,
               w1_ref, bf1_ref, w2_ref, bf2_ref, g2_ref, be2_ref, o_ref):
    src2 = (jnp.dot(at_ref[...], wo_ref[...],
                    preferred_element_type=jnp.float32) + bo_ref[...])
    s = src_ref[...] + src2
    mu = jnp.mean(s, axis=1, keepdims=True)
    var = jnp.mean((s - mu) ** 2, axis=1, keepdims=True)
    s = (s - mu) * jax.lax.rsqrt(var + 1e-5) * g1_ref[...] + be1_ref[...]
    f = jnp.maximum(jnp.dot(s, w1_ref[...],
                            preferred_element_type=jnp.float32)
                    + bf1_ref[...], 0.0)
    f = (jnp.dot(f, w2_ref[...], preferred_element_type=jnp.float32)
         + bf2_ref[...])
    o = s + f
    mu2 = jnp.mean(o, axis=1, keepdims=True)
    var2 = jnp.mean((o - mu2) ** 2, axis=1, keepdims=True)
    o_ref[...] = ((o - mu2) * jax.lax.rsqrt(var2 + 1e-5) * g2_ref[...]
                  + be2_ref[...])


def _run_tail(attn2d, src2d, Wo, bo, g1, be1, W1, bf1, W2, bf2, g2, be2):
    n = src2d.shape[0]
    grid = ((n + RB - 1) // RB,)
    row = lambda i: (i, 0)
    fix = lambda i: (0, 0)
    return pl.pallas_call(
        _tail_body,
        grid=grid,
        in_specs=[
            pl.BlockSpec((RB, D), row), pl.BlockSpec((RB, D), row),
            pl.BlockSpec((D, D), fix), pl.BlockSpec((1, D), fix),
            pl.BlockSpec((1, D), fix), pl.BlockSpec((1, D), fix),
            pl.BlockSpec((D, DFF), fix), pl.BlockSpec((1, DFF), fix),
            pl.BlockSpec((DFF, D), fix), pl.BlockSpec((1, D), fix),
            pl.BlockSpec((1, D), fix), pl.BlockSpec((1, D), fix),
        ],
        out_specs=pl.BlockSpec((RB, D), row),
        out_shape=jax.ShapeDtypeStruct((n, D), jnp.float32),
    )(attn2d, src2d, Wo, bo, g1, be1, W1, bf1, W2, bf2, g2, be2)


def kernel(src, pos, reference_points, spatial_shapes, level_start_index,
           window_grid, mask0, mask1, mask2, mask3, parse_w, parse_b,
           Wv, bv, Wo, bo, g1, be1, W1, bf1, W2, bf2, g2, be2):
    del pos, spatial_shapes, level_start_index, window_grid, parse_b
    masks = (mask0, mask1, mask2, mask3)

    svs, sis = [], []
    for l, (H, W) in enumerate(SHAPES):
        p = WS // 2
        mp = jnp.pad(masks[l][:, 0], ((0, 0), (p, p), (p, p)))
        sv, si = _run_topk(mp, H, W)
        svs.append(sv.reshape(BS, 2 * NP, H * W))
        sis.append(si.reshape(BS, 2 * NP, H * W))
    sv = jnp.concatenate(svs, axis=2)
    si = jnp.concatenate(sis, axis=2)

    refT = jnp.transpose(reference_points, (0, 2, 3, 1)).reshape(BS, 2 * NL, LQ)
    parse_wT = jnp.transpose(parse_w, (1, 0))
    gidx, coef = _run_coef(sv, si, refT, parse_wT)
    gidx_q = jnp.transpose(gidx, (0, 2, 1)).reshape(BS * LQ, NR)
    coef_q = jnp.transpose(coef, (0, 2, 1)).reshape(BS * LQ * NT)

    src2d = src.reshape(BS * LQ, D)
    perm = (jnp.arange(D).reshape(NH, 2, HD // 2).transpose(0, 2, 1)
            .reshape(D))
    val = _run_val(src2d, Wv[:, perm], bv[perm].reshape(1, D))
    val = val.astype(jnp.bfloat16).reshape(BS, LQ, NH, HD)
    parts = []
    for l, (H, W) in enumerate(SHAPES):
        PH, PW = PADS[l]
        v = val[:, START[l]:START[l + 1]].reshape(BS, H, W, NH, HD)
        v = jnp.transpose(v, (0, 3, 1, 2, 4))
        vp = jnp.pad(v, ((0, 0), (0, 0), (HALO, HALO + 1),
                         (HALO, HALO + 1), (0, 0)))
        quad = jnp.concatenate(
            [vp[:, :, :PH, :PW], vp[:, :, :PH, 1:PW + 1],
             vp[:, :, 1:PH + 1, :PW], vp[:, :, 1:PH + 1, 1:PW + 1]], axis=-1)
        parts.append(quad.reshape(BS * NH * PHW[l], 4 * HD))
    table = jnp.concatenate(parts, axis=0)

    attn_out = _sc_gather(table, gidx_q, coef_q)

    out = _run_tail(attn_out, src2d, Wo, bo.reshape(1, D),
                    g1.reshape(1, D), be1.reshape(1, D), W1,
                    bf1.reshape(1, DFF), W2, bf2.reshape(1, D),
                    g2.reshape(1, D), be2.reshape(1, D))
    return out.reshape(BS, LQ, D)


# speedup vs baseline: 1.0472x; 1.0472x over previous
import functools

import jax
import jax.numpy as jnp
from jax.experimental import pallas as pl
from jax.experimental.pallas import tpu as pltpu
from jax.experimental.pallas import tpu_sc as plsc

WS = 7
NH = 8
NP = 4
NL = 4
D = 256
HD = D // NH
DFF = 1024
BS = 2
SHAPES = ((64, 64), (32, 32), (16, 16), (8, 8))
LQ = sum(h * w for h, w in SHAPES)
START = (0, 4096, 5120, 5376, 5440)
HALO = 4
PADS = tuple((h + 2 * HALO, w + 2 * HALO) for h, w in SHAPES)
PHW = tuple(ph * pw for ph, pw in PADS)
LVL_BASE = (0,)
for _p in PHW:
    LVL_BASE = LVL_BASE + (LVL_BASE[-1] + BS * NH * _p,)
TOT_ROWS = LVL_BASE[-1]
NT = NH * NL * NP * 4
NR = NH * NL * NP
QB = 256
RB = 256


def _topk_body(mp_ref, sv_ref, si_ref, *, H, W, HB):
    base = pl.program_id(1) * HB
    sub = mp_ref[0, pl.ds(base, HB + 6), :]
    mx = [sub[:, dx:dx + W] for dx in range(WS)]
    ninf = jnp.full((HB, W), -jnp.inf, jnp.float32)
    zi = jnp.zeros((HB, W), jnp.int32)
    bv = [ninf] * (2 * NP)
    bi = [zi] * (2 * NP)
    for s in range(WS * WS):
        dy, dx = s // WS, s % WS
        if True:
            shift = mx[dx][dy:dy + HB, :]
            for sgn in range(2):
                v = shift if sgn == 0 else -shift
                i = jnp.full((HB, W), s, jnp.int32)
                for k in range(NP):
                    o = sgn * NP + k
                    gt = v > bv[o]
                    bv[o], v = jnp.where(gt, v, bv[o]), jnp.where(gt, bv[o], v)
                    bi[o], i = jnp.where(gt, i, bi[o]), jnp.where(gt, bi[o], i)
    sv_ref[...] = jnp.stack(bv, axis=0)[None]
    si_ref[...] = jnp.stack(bi, axis=0)[None]


def _run_topk(mp, H, W):
    HB = min(H, 16)
    grid = (BS, H // HB)
    return pl.pallas_call(
        functools.partial(_topk_body, H=H, W=W, HB=HB),
        grid=grid,
        in_specs=[pl.BlockSpec((1, H + 6, W + 6), lambda b, j: (b, 0, 0))],
        out_specs=(pl.BlockSpec((1, 2 * NP, HB, W), lambda b, j: (b, 0, j, 0)),
                   pl.BlockSpec((1, 2 * NP, HB, W), lambda b, j: (b, 0, j, 0))),
        out_shape=(jax.ShapeDtypeStruct((BS, 2 * NP, H, W), jnp.float32),
                   jax.ShapeDtypeStruct((BS, 2 * NP, H, W), jnp.int32)),
    )(mp)


def _coef_body(sv_ref, si_ref, ref_ref, pw_ref, gidx_ref, coef_ref):
    b = pl.program_id(0)
    j = pl.program_id(1)
    lam = ((j >= 16).astype(jnp.int32) + (j >= 20).astype(jnp.int32)
           + (j >= 21).astype(jnp.int32))
    wsel = jnp.zeros((NH, 1), jnp.float32)
    for l in range(NL):
        wsel = jnp.where(lam == l, pw_ref[:, l:l + 1], wsel)
    pos = wsel >= 0.0
    sv = sv_ref[0]
    si = si_ref[0]
    vals = jnp.where(pos[:, :, None], sv[None, 0:NP], sv[None, NP:2 * NP])
    idx = jnp.where(pos[:, :, None], si[None, 0:NP], si[None, NP:2 * NP])
    z = jnp.abs(wsel)[:, :, None] * vals
    zm = jnp.maximum(jnp.maximum(z[:, 0], z[:, 1]),
                     jnp.maximum(z[:, 2], z[:, 3]))
    e = jnp.exp(z - zm[:, None, :])
    es = e[:, 0] + e[:, 1] + e[:, 2] + e[:, 3]
    attn = e / (NL * es)[:, None, :]
    ref2 = ref_ref[0]
    hiota = jax.lax.broadcasted_iota(jnp.int32, (NH, 1, 1), 0)
    rows_l = []
    coef_l = []
    for l, (H, W) in enumerate(SHAPES):
        PH, PW = PADS[l]
        x = ref2[2 * l] * W - 0.5
        y = ref2[2 * l + 1] * H - 0.5
        x0f = jnp.floor(x)
        y0f = jnp.floor(y)
        lx = x - x0f
        ly = y - y0f
        off = (idx // WS) * PW + (idx % WS)
        qbase = (y0f.astype(jnp.int32) * PW + x0f.astype(jnp.int32)
                 + (LVL_BASE[l] + b * NH * PHW[l]
                    + (HALO - WS // 2) * PW + (HALO - WS // 2)))
        r00 = off + qbase[None, None, :] + hiota * PHW[l]
        rows_l.append(jnp.clip(r00, 0, TOT_ROWS - 1))
        ca = attn
        coef_l.append(jnp.stack(
            [ca * ((1 - lx) * (1 - ly))[None, None, :],
             ca * (lx * (1 - ly))[None, None, :],
             ca * ((1 - lx) * ly)[None, None, :],
             ca * (lx * ly)[None, None, :]], axis=2))
    rows = jnp.stack(rows_l, axis=1)
    coefs = jnp.stack(coef_l, axis=1)
    gidx_ref[...] = rows.reshape(1, NR, QB)
    coef_ref[...] = coefs.reshape(1, NT, QB)


def _run_coef(sv, si, refT, parse_wT):
    nblk = (LQ + QB - 1) // QB
    grid = (BS, nblk)
    return pl.pallas_call(
        _coef_body,
        grid=grid,
        in_specs=[
            pl.BlockSpec((1, 2 * NP, QB), lambda b, j: (b, 0, j)),
            pl.BlockSpec((1, 2 * NP, QB), lambda b, j: (b, 0, j)),
            pl.BlockSpec((1, 2 * NL, QB), lambda b, j: (b, 0, j)),
            pl.BlockSpec((NH, NL), lambda b, j: (0, 0)),
        ],
        out_specs=(
            pl.BlockSpec((1, NR, QB), lambda b, j: (b, 0, j)),
            pl.BlockSpec((1, NT, QB), lambda b, j: (b, 0, j)),
        ),
        out_shape=(jax.ShapeDtypeStruct((BS, NR, LQ), jnp.int32),
                   jax.ShapeDtypeStruct((BS, NT, LQ), jnp.float32)),
    )(sv, si, refT, parse_wT)


def _val_body(x_ref, w_ref, b_ref, o_ref):
    o_ref[...] = (jnp.dot(x_ref[...], w_ref[...],
                          preferred_element_type=jnp.float32) + b_ref[...])


def _run_val(src2d, Wv, bv):
    n = src2d.shape[0]
    grid = ((n + RB - 1) // RB,)
    return pl.pallas_call(
        _val_body,
        grid=grid,
        in_specs=[
            pl.BlockSpec((RB, D), lambda i: (i, 0)),
            pl.BlockSpec((D, D), lambda i: (0, 0)),
            pl.BlockSpec((1, D), lambda i: (0, 0)),
        ],
        out_specs=pl.BlockSpec((RB, D), lambda i: (i, 0)),
        out_shape=jax.ShapeDtypeStruct((n, D), jnp.float32),
    )(src2d, Wv, bv)


NWORK = 32
QPW = BS * LQ // NWORK
CQ = 4
NCHUNK = QPW // CQ


CN = CQ * NT
CR = CQ * NR


def _sc_body(table, idxr, coefr, out_hbm, idxv, coefv, rowsv, outv,
             sem, seml):
    cid = jax.lax.axis_index("c")
    sid = jax.lax.axis_index("s")
    wid = sid * 2 + cid
    q0w = wid * QPW

    def load_idx(t, pb):
        q0 = q0w + t * CQ
        pltpu.async_copy(idxr.at[pl.ds(q0, CQ)],
                         idxv.at[pl.ds(pb * CQ, CQ)], seml)
        pltpu.async_copy(coefr.at[pl.ds(q0 * NT, CN)],
                         coefv.at[pl.ds(pb * CN, CN)], seml)

    def wait_idx(t, pb):
        q0 = q0w + t * CQ
        pltpu.make_async_copy(idxr.at[pl.ds(q0, CQ)],
                              idxv.at[pl.ds(pb * CQ, CQ)], seml).wait()
        pltpu.make_async_copy(coefr.at[pl.ds(q0 * NT, CN)],
                              coefv.at[pl.ds(pb * CN, CN)], seml).wait()

    def fire(pb):
        for jj in range(CQ):
            pltpu.async_copy(
                table.at[idxv.at[pb * CQ + jj]],
                rowsv.at[pl.ds(pb * CR + jj * NR, NR)], sem)

    def drain(pb):
        for jj in range(CQ):
            pltpu.make_async_copy(
                table.at[idxv.at[pb * CQ + jj]],
                rowsv.at[pl.ds(pb * CR + jj * NR, NR)], sem).wait()

    load_idx(0, 0)
    wait_idx(0, 0)
    fire(0)
    load_idx(jnp.int32(min(1, NCHUNK - 1)), 1)

    def chunk(t, carry):
        pb = jax.lax.rem(t, 2)
        drain(pb)
        tn = jnp.minimum(t + 1, NCHUNK - 1)
        wait_idx(tn, 1 - pb)
        fire(1 - pb)
        cb = pb * CN
        rb0 = pb * CR
        for qq in range(CQ):
            for h in range(NH):
                def body(bk, ac, qq=qq, h=h, cb=cb, rb0=rb0):
                    ac = list(ac)
                    cbase = cb + qq * NT + h * 64 + bk * 16
                    rbase = rb0 + qq * NR + h * 16 + bk * 4
                    for j2 in range(4):
                        for cc in range(4):
                            c = plsc.load_gather(
                                coefv, [jnp.zeros((16,), jnp.int32)
                                        + (cbase + j2 * 4 + cc)])
                            w = plsc.bitcast(
                                rowsv[rbase + j2, pl.ds(cc * HD, HD)],
                                jnp.int32)
                            r0 = plsc.bitcast(w << 16, jnp.float32)
                            r1 = plsc.bitcast(w & jnp.int32(-65536),
                                              jnp.float32)
                            ac[2 * cc] = ac[2 * cc] + c * r0
                            ac[2 * cc + 1] = ac[2 * cc + 1] + c * r1
                    return tuple(ac)

                z16 = jnp.zeros((16,), jnp.float32)
                acc = jax.lax.fori_loop(0, 4, body, (z16,) * 8)
                outv[pb * CQ + qq, pl.ds(h * HD, 16)] = (
                    (acc[0] + acc[2]) + (acc[4] + acc[6]))
                outv[pb * CQ + qq, pl.ds(h * HD + 16, 16)] = (
                    (acc[1] + acc[3]) + (acc[5] + acc[7]))
        load_idx(jnp.minimum(t + 2, NCHUNK - 1), pb)
        pltpu.sync_copy(outv.at[pl.ds(pb * CQ, CQ)],
                        out_hbm.at[pl.ds(q0w + t * CQ, CQ)])
        return carry

    jax.lax.fori_loop(0, NCHUNK, chunk, 0)
    drain(NCHUNK % 2)
    wait_idx(jnp.int32(NCHUNK - 1), (NCHUNK - 1) % 2)


@functools.lru_cache(maxsize=1)
def _sc_gather_fn():
    return functools.partial(
        pl.kernel,
        out_type=jax.ShapeDtypeStruct((BS * LQ, D), jnp.float32),
        mesh=plsc.VectorSubcoreMesh(core_axis_name="c", subcore_axis_name="s"),
        compiler_params=pltpu.CompilerParams(use_tc_tiling_on_sc=False,
                                             needs_layout_passes=False),
        scratch_types=[
            pltpu.VMEM((2 * CQ, NR), jnp.int32),
            pltpu.VMEM((2 * CN,), jnp.float32),
            pltpu.VMEM((2 * CR, 4 * HD), jnp.bfloat16),
            pltpu.VMEM((2 * CQ, D), jnp.float32),
            pltpu.SemaphoreType.DMA,
            pltpu.SemaphoreType.DMA,
        ],
    )(_sc_body)


def _sc_gather(table, gidx_q, coef_q):
    return _sc_gather_fn()(table, gidx_q, coef_q)


def _tail_body(at_ref, src_ref, wo_ref, bo_ref, g1_ref, be1_ref,
               w1_ref, bf1_ref, w2_ref, bf2_ref, g2_ref, be2_ref, o_ref):
    src2 = (jnp.dot(at_ref[...], wo_ref[...],
                    preferred_element_type=jnp.float32) + bo_ref[...])
    s = src_ref[...] + src2
    mu = jnp.mean(s, axis=1, keepdims=True)
    var = jnp.mean((s - mu) ** 2, axis=1, keepdims=True)
    s = (s - mu) * jax.lax.rsqrt(var + 1e-5) * g1_ref[...] + be1_ref[...]
    f = jnp.maximum(jnp.dot(s, w1_ref[...],
                            preferred_element_type=jnp.float32)
                    + bf1_ref[...], 0.0)
    f = (jnp.dot(f, w2_ref[...], preferred_element_type=jnp.float32)
         + bf2_ref[...])
    o = s + f
    mu2 = jnp.mean(o, axis=1, keepdims=True)
    var2 = jnp.mean((o - mu2) ** 2, axis=1, keepdims=True)
    o_ref[...] = ((o - mu2) * jax.lax.rsqrt(var2 + 1e-5) * g2_ref[...]
                  + be2_ref[...])


def _run_tail(attn2d, src2d, Wo, bo, g1, be1, W1, bf1, W2, bf2, g2, be2):
    n = src2d.shape[0]
    grid = ((n + RB - 1) // RB,)
    row = lambda i: (i, 0)
    fix = lambda i: (0, 0)
    return pl.pallas_call(
        _tail_body,
        grid=grid,
        in_specs=[
            pl.BlockSpec((RB, D), row), pl.BlockSpec((RB, D), row),
            pl.BlockSpec((D, D), fix), pl.BlockSpec((1, D), fix),
            pl.BlockSpec((1, D), fix), pl.BlockSpec((1, D), fix),
            pl.BlockSpec((D, DFF), fix), pl.BlockSpec((1, DFF), fix),
            pl.BlockSpec((DFF, D), fix), pl.BlockSpec((1, D), fix),
            pl.BlockSpec((1, D), fix), pl.BlockSpec((1, D), fix),
        ],
        out_specs=pl.BlockSpec((RB, D), row),
        out_shape=jax.ShapeDtypeStruct((n, D), jnp.float32),
    )(attn2d, src2d, Wo, bo, g1, be1, W1, bf1, W2, bf2, g2, be2)


def kernel(src, pos, reference_points, spatial_shapes, level_start_index,
           window_grid, mask0, mask1, mask2, mask3, parse_w, parse_b,
           Wv, bv, Wo, bo, g1, be1, W1, bf1, W2, bf2, g2, be2):
    del pos, spatial_shapes, level_start_index, window_grid, parse_b
    masks = (mask0, mask1, mask2, mask3)

    svs, sis = [], []
    for l, (H, W) in enumerate(SHAPES):
        p = WS // 2
        mp = jnp.pad(masks[l][:, 0], ((0, 0), (p, p), (p, p)))
        sv, si = _run_topk(mp, H, W)
        svs.append(sv.reshape(BS, 2 * NP, H * W))
        sis.append(si.reshape(BS, 2 * NP, H * W))
    sv = jnp.concatenate(svs, axis=2)
    si = jnp.concatenate(sis, axis=2)

    refT = jnp.transpose(reference_points, (0, 2, 3, 1)).reshape(BS, 2 * NL, LQ)
    parse_wT = jnp.transpose(parse_w, (1, 0))
    gidx, coef = _run_coef(sv, si, refT, parse_wT)
    gidx_q = jnp.transpose(gidx, (0, 2, 1)).reshape(BS * LQ, NR)
    coef_q = jnp.transpose(coef, (0, 2, 1)).reshape(BS * LQ * NT)

    src2d = src.reshape(BS * LQ, D)
    perm = (jnp.arange(D).reshape(NH, 2, HD // 2).transpose(0, 2, 1)
            .reshape(D))
    val = _run_val(src2d, Wv[:, perm], bv[perm].reshape(1, D))
    val = val.astype(jnp.bfloat16).reshape(BS, LQ, NH, HD)
    parts = []
    for l, (H, W) in enumerate(SHAPES):
        PH, PW = PADS[l]
        v = val[:, START[l]:START[l + 1]].reshape(BS, H, W, NH, HD)
        v = jnp.transpose(v, (0, 3, 1, 2, 4))
        vp = jnp.pad(v, ((0, 0), (0, 0), (HALO, HALO + 1),
                         (HALO, HALO + 1), (0, 0)))
        quad = jnp.concatenate(
            [vp[:, :, :PH, :PW], vp[:, :, :PH, 1:PW + 1],
             vp[:, :, 1:PH + 1, :PW], vp[:, :, 1:PH + 1, 1:PW + 1]], axis=-1)
        parts.append(quad.reshape(BS * NH * PHW[l], 4 * HD))
    table = jnp.concatenate(parts, axis=0)

    attn_out = _sc_gather(table, gidx_q, coef_q)

    out = _run_tail(attn_out, src2d, Wo, bo.reshape(1, D),
                    g1.reshape(1, D), be1.reshape(1, D), W1,
                    bf1.reshape(1, DFF), W2, bf2.reshape(1, D),
                    g2.reshape(1, D), be2.reshape(1, D))
    return out.reshape(BS, LQ, D)


# speedup vs baseline: 1.1331x; 1.0820x over previous
import functools

import jax
import jax.numpy as jnp
from jax.experimental import pallas as pl
from jax.experimental.pallas import tpu as pltpu
from jax.experimental.pallas import tpu_sc as plsc

WS = 7
NH = 8
NP = 4
NL = 4
D = 256
HD = D // NH
DFF = 1024
BS = 2
SHAPES = ((64, 64), (32, 32), (16, 16), (8, 8))
LQ = sum(h * w for h, w in SHAPES)
START = (0, 4096, 5120, 5376, 5440)
HALO = 4
PADS = tuple((h + 2 * HALO, w + 2 * HALO) for h, w in SHAPES)
PHW = tuple(ph * pw for ph, pw in PADS)
LVL_BASE = (0,)
for _p in PHW:
    LVL_BASE = LVL_BASE + (LVL_BASE[-1] + BS * NH * _p,)
TOT_ROWS = LVL_BASE[-1]
NT = NH * NL * NP * 4
NR = NH * NL * NP
QB = 256
RB = 256


def _topk_body(mp_ref, sv_ref, si_ref, *, H, W, HB):
    base = pl.program_id(1) * HB
    sub = mp_ref[0, pl.ds(base, HB + 6), :]
    mx = [sub[:, dx:dx + W] for dx in range(WS)]
    ninf = jnp.full((HB, W), -jnp.inf, jnp.float32)
    zi = jnp.zeros((HB, W), jnp.int32)
    bv = [ninf] * (2 * NP)
    bi = [zi] * (2 * NP)
    for s in range(WS * WS):
        dy, dx = s // WS, s % WS
        if True:
            shift = mx[dx][dy:dy + HB, :]
            for sgn in range(2):
                v = shift if sgn == 0 else -shift
                i = jnp.full((HB, W), s, jnp.int32)
                for k in range(NP):
                    o = sgn * NP + k
                    gt = v > bv[o]
                    bv[o], v = jnp.where(gt, v, bv[o]), jnp.where(gt, bv[o], v)
                    bi[o], i = jnp.where(gt, i, bi[o]), jnp.where(gt, bi[o], i)
    sv_ref[...] = jnp.stack(bv, axis=0)[None]
    si_ref[...] = jnp.stack(bi, axis=0)[None]


def _run_topk(mp, H, W):
    HB = min(H, 16)
    grid = (BS, H // HB)
    return pl.pallas_call(
        functools.partial(_topk_body, H=H, W=W, HB=HB),
        grid=grid,
        in_specs=[pl.BlockSpec((1, H + 6, W + 6), lambda b, j: (b, 0, 0))],
        out_specs=(pl.BlockSpec((1, 2 * NP, HB, W), lambda b, j: (b, 0, j, 0)),
                   pl.BlockSpec((1, 2 * NP, HB, W), lambda b, j: (b, 0, j, 0))),
        out_shape=(jax.ShapeDtypeStruct((BS, 2 * NP, H, W), jnp.float32),
                   jax.ShapeDtypeStruct((BS, 2 * NP, H, W), jnp.int32)),
    )(mp)


def _coef_body(sv_ref, si_ref, ref_ref, pw_ref, gidx_ref, coef_ref):
    b = pl.program_id(0)
    j = pl.program_id(1)
    lam = ((j >= 16).astype(jnp.int32) + (j >= 20).astype(jnp.int32)
           + (j >= 21).astype(jnp.int32))
    wsel = jnp.zeros((NH, 1), jnp.float32)
    for l in range(NL):
        wsel = jnp.where(lam == l, pw_ref[:, l:l + 1], wsel)
    pos = wsel >= 0.0
    sv = sv_ref[0]
    si = si_ref[0]
    vals = jnp.where(pos[:, :, None], sv[None, 0:NP], sv[None, NP:2 * NP])
    idx = jnp.where(pos[:, :, None], si[None, 0:NP], si[None, NP:2 * NP])
    z = jnp.abs(wsel)[:, :, None] * vals
    zm = jnp.maximum(jnp.maximum(z[:, 0], z[:, 1]),
                     jnp.maximum(z[:, 2], z[:, 3]))
    e = jnp.exp(z - zm[:, None, :])
    es = e[:, 0] + e[:, 1] + e[:, 2] + e[:, 3]
    attn = e / (NL * es)[:, None, :]
    ref2 = ref_ref[0]
    hiota = jax.lax.broadcasted_iota(jnp.int32, (NH, 1, 1), 0)
    rows_l = []
    coef_l = []
    for l, (H, W) in enumerate(SHAPES):
        PH, PW = PADS[l]
        x = ref2[2 * l] * W - 0.5
        y = ref2[2 * l + 1] * H - 0.5
        x0f = jnp.floor(x)
        y0f = jnp.floor(y)
        lx = x - x0f
        ly = y - y0f
        off = (idx // WS) * PW + (idx % WS)
        qbase = (y0f.astype(jnp.int32) * PW + x0f.astype(jnp.int32)
                 + (LVL_BASE[l] + b * NH * PHW[l]
                    + (HALO - WS // 2) * PW + (HALO - WS // 2)))
        r00 = off + qbase[None, None, :] + hiota * PHW[l]
        r00 = jnp.clip(r00, 0, TOT_ROWS - PW - 2)
        rows_l.append(jnp.stack(
            [r00, r00 + 1, r00 + PW, r00 + PW + 1], axis=2))
        ca = attn
        coef_l.append(jnp.stack(
            [ca * ((1 - lx) * (1 - ly))[None, None, :],
             ca * (lx * (1 - ly))[None, None, :],
             ca * ((1 - lx) * ly)[None, None, :],
             ca * (lx * ly)[None, None, :]], axis=2))
    rows = jnp.stack(rows_l, axis=1)
    coefs = jnp.stack(coef_l, axis=1)
    gidx_ref[...] = rows.reshape(1, NT, QB)
    coef_ref[...] = coefs.reshape(1, NT, QB)


def _run_coef(sv, si, refT, parse_wT):
    nblk = (LQ + QB - 1) // QB
    grid = (BS, nblk)
    return pl.pallas_call(
        _coef_body,
        grid=grid,
        in_specs=[
            pl.BlockSpec((1, 2 * NP, QB), lambda b, j: (b, 0, j)),
            pl.BlockSpec((1, 2 * NP, QB), lambda b, j: (b, 0, j)),
            pl.BlockSpec((1, 2 * NL, QB), lambda b, j: (b, 0, j)),
            pl.BlockSpec((NH, NL), lambda b, j: (0, 0)),
        ],
        out_specs=(
            pl.BlockSpec((1, NT, QB), lambda b, j: (b, 0, j)),
            pl.BlockSpec((1, NT, QB), lambda b, j: (b, 0, j)),
        ),
        out_shape=(jax.ShapeDtypeStruct((BS, NT, LQ), jnp.int32),
                   jax.ShapeDtypeStruct((BS, NT, LQ), jnp.float32)),
    )(sv, si, refT, parse_wT)


def _val_body(x_ref, w_ref, b_ref, o_ref):
    o_ref[...] = (jnp.dot(x_ref[...], w_ref[...],
                          preferred_element_type=jnp.float32) + b_ref[...])


def _run_val(src2d, Wv, bv):
    n = src2d.shape[0]
    grid = ((n + RB - 1) // RB,)
    return pl.pallas_call(
        _val_body,
        grid=grid,
        in_specs=[
            pl.BlockSpec((RB, D), lambda i: (i, 0)),
            pl.BlockSpec((D, D), lambda i: (0, 0)),
            pl.BlockSpec((1, D), lambda i: (0, 0)),
        ],
        out_specs=pl.BlockSpec((RB, D), lambda i: (i, 0)),
        out_shape=jax.ShapeDtypeStruct((n, D), jnp.float32),
    )(src2d, Wv, bv)


NWORK = 32
QPW = BS * LQ // NWORK
CQ = 2
NCHUNK = QPW // CQ


CN = CQ * NT
CR = CQ * NR


def _sc_body(table, idxr, coefr, out_hbm, idxv, coefv, rowsv, outv, sem):
    cid = jax.lax.axis_index("c")
    sid = jax.lax.axis_index("s")
    wid = sid * 2 + cid
    q0w = wid * QPW

    def load_fire(t, pb):
        q0 = q0w + t * CQ
        pltpu.sync_copy(idxr.at[pl.ds(q0 * 4, 4 * CQ)],
                        idxv.at[pl.ds(pb * 4 * CQ, 4 * CQ)])
        pltpu.sync_copy(coefr.at[pl.ds(q0 * NT, CN)],
                        coefv.at[pl.ds(pb * CN, CN)])
        for jj in range(4 * CQ):
            pltpu.async_copy(
                table.at[idxv.at[pb * 4 * CQ + jj]],
                rowsv.at[pl.ds(pb * CN + jj * 128, 128)], sem)

    def drain(pb):
        for jj in range(4 * CQ):
            pltpu.make_async_copy(
                table.at[idxv.at[pb * 4 * CQ + jj]],
                rowsv.at[pl.ds(pb * CN + jj * 128, 128)], sem).wait()

    load_fire(0, 0)

    def chunk(t, carry):
        pb = jax.lax.rem(t, 2)
        drain(pb)
        tn = jnp.minimum(t + 1, NCHUNK - 1)
        load_fire(tn, 1 - pb)
        cb = pb * CN
        for qq in range(CQ):
            for h in range(NH):
                def body(bk, ac, qq=qq, h=h, cb=cb):
                    a0, a1 = ac
                    base = cb + qq * NT + h * 64 + bk * 16
                    for j2 in range(16):
                        c = plsc.load_gather(
                            coefv, [jnp.zeros((16,), jnp.int32) + (base + j2)])
                        a0 = a0 + c * rowsv[base + j2, pl.ds(0, 16)]
                        a1 = a1 + c * rowsv[base + j2, pl.ds(16, 16)]
                    return (a0, a1)

                z16 = jnp.zeros((16,), jnp.float32)
                a0, a1 = jax.lax.fori_loop(0, 4, body, (z16, z16))
                outv[pb * CQ + qq, pl.ds(h * HD, 16)] = a0
                outv[pb * CQ + qq, pl.ds(h * HD + 16, 16)] = a1
        pltpu.sync_copy(outv.at[pl.ds(pb * CQ, CQ)],
                        out_hbm.at[pl.ds(q0w + t * CQ, CQ)])
        return carry

    jax.lax.fori_loop(0, NCHUNK, chunk, 0)
    drain(0)


@functools.lru_cache(maxsize=1)
def _sc_gather_fn():
    return functools.partial(
        pl.kernel,
        out_type=jax.ShapeDtypeStruct((BS * LQ, D), jnp.float32),
        mesh=plsc.VectorSubcoreMesh(core_axis_name="c", subcore_axis_name="s"),
        compiler_params=pltpu.CompilerParams(use_tc_tiling_on_sc=False,
                                             needs_layout_passes=False),
        scratch_types=[
            pltpu.VMEM((2 * 4 * CQ, 128), jnp.int32),
            pltpu.VMEM((2 * CN,), jnp.float32),
            pltpu.VMEM((2 * CN, HD), jnp.float32),
            pltpu.VMEM((2 * CQ, D), jnp.float32),
            pltpu.SemaphoreType.DMA,
        ],
    )(_sc_body)


def _sc_gather(table, gidx_q, coef_q):
    return _sc_gather_fn()(table, gidx_q, coef_q)


def _tail_body(at_ref, src_ref, wo_ref, bo_ref, g1_ref, be1_ref,
               w1_ref, bf1_ref, w2_ref, bf2_ref, g2_ref, be2_ref, o_ref):
    src2 = (jnp.dot(at_ref[...], wo_ref[...],
                    preferred_element_type=jnp.float32) + bo_ref[...])
    s = src_ref[...] + src2
    mu = jnp.mean(s, axis=1, keepdims=True)
    var = jnp.mean((s - mu) ** 2, axis=1, keepdims=True)
    s = (s - mu) * jax.lax.rsqrt(var + 1e-5) * g1_ref[...] + be1_ref[...]
    f = jnp.maximum(jnp.dot(s, w1_ref[...],
                            preferred_element_type=jnp.float32)
                    + bf1_ref[...], 0.0)
    f = (jnp.dot(f, w2_ref[...], preferred_element_type=jnp.float32)
         + bf2_ref[...])
    o = s + f
    mu2 = jnp.mean(o, axis=1, keepdims=True)
    var2 = jnp.mean((o - mu2) ** 2, axis=1, keepdims=True)
    o_ref[...] = ((o - mu2) * jax.lax.rsqrt(var2 + 1e-5) * g2_ref[...]
                  + be2_ref[...])


def _run_tail(attn2d, src2d, Wo, bo, g1, be1, W1, bf1, W2, bf2, g2, be2):
    n = src2d.shape[0]
    grid = ((n + RB - 1) // RB,)
    row = lambda i: (i, 0)
    fix = lambda i: (0, 0)
    return pl.pallas_call(
        _tail_body,
        grid=grid,
        in_specs=[
            pl.BlockSpec((RB, D), row), pl.BlockSpec((RB, D), row),
            pl.BlockSpec((D, D), fix), pl.BlockSpec((1, D), fix),
            pl.BlockSpec((1, D), fix), pl.BlockSpec((1, D), fix),
            pl.BlockSpec((D, DFF), fix), pl.BlockSpec((1, DFF), fix),
            pl.BlockSpec((DFF, D), fix), pl.BlockSpec((1, D), fix),
            pl.BlockSpec((1, D), fix), pl.BlockSpec((1, D), fix),
        ],
        out_specs=pl.BlockSpec((RB, D), row),
        out_shape=jax.ShapeDtypeStruct((n, D), jnp.float32),
    )(attn2d, src2d, Wo, bo, g1, be1, W1, bf1, W2, bf2, g2, be2)


def kernel(src, pos, reference_points, spatial_shapes, level_start_index,
           window_grid, mask0, mask1, mask2, mask3, parse_w, parse_b,
           Wv, bv, Wo, bo, g1, be1, W1, bf1, W2, bf2, g2, be2):
    del pos, spatial_shapes, level_start_index, window_grid, parse_b
    masks = (mask0, mask1, mask2, mask3)

    svs, sis = [], []
    for l, (H, W) in enumerate(SHAPES):
        p = WS // 2
        mp = jnp.pad(masks[l][:, 0], ((0, 0), (p, p), (p, p)))
        sv, si = _run_topk(mp, H, W)
        svs.append(sv.reshape(BS, 2 * NP, H * W))
        sis.append(si.reshape(BS, 2 * NP, H * W))
    sv = jnp.concatenate(svs, axis=2)
    si = jnp.concatenate(sis, axis=2)

    refT = jnp.transpose(reference_points, (0, 2, 3, 1)).reshape(BS, 2 * NL, LQ)
    parse_wT = jnp.transpose(parse_w, (1, 0))
    gidx, coef = _run_coef(sv, si, refT, parse_wT)
    gidx_q = jnp.transpose(gidx, (0, 2, 1)).reshape(BS * LQ * 4, 128)
    coef_q = jnp.transpose(coef, (0, 2, 1)).reshape(BS * LQ * NT)

    src2d = src.reshape(BS * LQ, D)
    val = _run_val(src2d, Wv, bv.reshape(1, D)).reshape(BS, LQ, NH, HD)
    parts = []
    for l, (H, W) in enumerate(SHAPES):
        v = val[:, START[l]:START[l + 1]].reshape(BS, H, W, NH, HD)
        v = jnp.transpose(v, (0, 3, 1, 2, 4))
        v = jnp.pad(v, ((0, 0), (0, 0), (HALO, HALO), (HALO, HALO), (0, 0)))
        parts.append(v.reshape(BS * NH * PHW[l], HD))
    table = jnp.concatenate(parts, axis=0)

    attn_out = _sc_gather(table, gidx_q, coef_q)

    out = _run_tail(attn_out, src2d, Wo, bo.reshape(1, D),
                    g1.reshape(1, D), be1.reshape(1, D), W1,
                    bf1.reshape(1, DFF), W2, bf2.reshape(1, D),
                    g2.reshape(1, D), be2.reshape(1, D))
    return out.reshape(BS, LQ, D)
